# baseline (speedup 1.0000x reference)
import jax
import jax.numpy as jnp
from jax.experimental import pallas as pl
from jax.experimental.pallas import tpu as pltpu

_PACK = 8
_BLOCK_GROUPS = 2048


def _packed_mlp_kernel(x_ref, w1p_ref, b1p_ref, w2p_ref, b2p_ref, o_ref):
    h = jnp.dot(x_ref[...], w1p_ref[...], preferred_element_type=jnp.float32)
    h = jnp.maximum(h + b1p_ref[...], 0.0)
    y = jnp.dot(h, w2p_ref[...], preferred_element_type=jnp.float32)
    o_ref[...] = (y + b2p_ref[...]).astype(o_ref.dtype)


def kernel(x, w1, b1, w2, b2):
    B, d_in = x.shape
    d_hidden = w1.shape[1]
    d_out = w2.shape[1]
    P = _PACK

    eye = jnp.eye(P, dtype=jnp.float32)
    w1p = jnp.kron(eye, w1.astype(jnp.float32))
    b1p = jnp.tile(b1.astype(jnp.float32), (1, P))
    w2p = jnp.kron(eye, w2.astype(jnp.float32))
    b2p = jnp.tile(b2.astype(jnp.float32), (1, P))

    G = B // P
    xp = x.reshape(G, P * d_in)

    gb = _BLOCK_GROUPS
    while G % gb != 0:
        gb //= 2
    grid = (G // gb,)

    vmem = pltpu.MemorySpace.VMEM
    outp = pl.pallas_call(
        _packed_mlp_kernel,
        out_shape=jax.ShapeDtypeStruct((G, P * d_out), x.dtype),
        grid=grid,
        in_specs=[
            pl.BlockSpec((gb, P * d_in), lambda i: (i, 0), memory_space=vmem),
            pl.BlockSpec((P * d_in, P * d_hidden), lambda i: (0, 0), memory_space=vmem),
            pl.BlockSpec((1, P * d_hidden), lambda i: (0, 0), memory_space=vmem),
            pl.BlockSpec((P * d_hidden, P * d_out), lambda i: (0, 0), memory_space=vmem),
            pl.BlockSpec((1, P * d_out), lambda i: (0, 0), memory_space=vmem),
        ],
        out_specs=pl.BlockSpec((gb, P * d_out), lambda i: (i, 0), memory_space=vmem),
        compiler_params=pltpu.CompilerParams(
            dimension_semantics=("parallel",),
        ),
    )(xp, w1p, b1p, w2p, b2p)

    return outp.reshape(B, d_out)


# speedup vs baseline: 1.5350x; 1.5350x over previous
import jax
import jax.numpy as jnp
from jax.experimental import pallas as pl
from jax.experimental.pallas import tpu as pltpu

_PACK = 8
_BLOCK_ROWS = 16384


def _packed_mlp_kernel(x_ref, w1p_ref, b1p_ref, w2p_ref, b2p_ref, o_ref):
    g = x_ref.shape[0]
    kin = x_ref.shape[1] * x_ref.shape[2]
    xp = x_ref[...].reshape(g, kin)
    h = jnp.dot(xp, w1p_ref[...], preferred_element_type=jnp.float32)
    h = jnp.maximum(h + b1p_ref[...], 0.0)
    y = jnp.dot(h, w2p_ref[...], preferred_element_type=jnp.float32)
    y = y + b2p_ref[...]
    o_ref[...] = y.reshape(o_ref.shape).astype(o_ref.dtype)


def kernel(x, w1, b1, w2, b2):
    B, d_in = x.shape
    d_hidden = w1.shape[1]
    d_out = w2.shape[1]
    P = _PACK

    eye = jnp.eye(P, dtype=jnp.float32)
    w1p = jnp.kron(eye, w1.astype(jnp.float32))
    b1p = jnp.tile(b1.astype(jnp.float32), (1, P))
    w2p = jnp.kron(eye, w2.astype(jnp.float32))
    b2p = jnp.tile(b2.astype(jnp.float32), (1, P))

    G = B // P
    x3 = x.reshape(G, P, d_in)

    tb = _BLOCK_ROWS
    while B % tb != 0:
        tb //= 2
    gb = tb // P
    grid = (B // tb,)

    vmem = pltpu.MemorySpace.VMEM
    out3 = pl.pallas_call(
        _packed_mlp_kernel,
        out_shape=jax.ShapeDtypeStruct((G, P, d_out), x.dtype),
        grid=grid,
        in_specs=[
            pl.BlockSpec((gb, P, d_in), lambda i: (i, 0, 0), memory_space=vmem),
            pl.BlockSpec((P * d_in, P * d_hidden), lambda i: (0, 0), memory_space=vmem),
            pl.BlockSpec((1, P * d_hidden), lambda i: (0, 0), memory_space=vmem),
            pl.BlockSpec((P * d_hidden, P * d_out), lambda i: (0, 0), memory_space=vmem),
            pl.BlockSpec((1, P * d_out), lambda i: (0, 0), memory_space=vmem),
        ],
        out_specs=pl.BlockSpec((gb, P, d_out), lambda i: (i, 0, 0), memory_space=vmem),
        compiler_params=pltpu.CompilerParams(
            dimension_semantics=("parallel",),
        ),
    )(x3, w1p, b1p, w2p, b2p)

    return out3.reshape(B, d_out)
